# baseline (speedup 1.0000x reference)
import functools

import jax
import jax.numpy as jnp
from jax import lax
from jax.experimental import pallas as pl
from jax.experimental.pallas import tpu as pltpu
from jax.experimental.pallas import tpu_sc as plsc

_B = 128
_N = 4096
_T = 2.0
_NC = 2
_NS = 16
_NW = _NC * _NS
_BPW = _B // _NW
_NA = 9
_VPR = _N // 16
_NEG = float("-inf")


@functools.partial(
    pl.kernel,
    mesh=plsc.VectorSubcoreMesh(core_axis_name="c", subcore_axis_name="s"),
    out_type=[
        jax.ShapeDtypeStruct((_B * 16,), jnp.float32),
        jax.ShapeDtypeStruct((_NW * 16,), jnp.float32),
    ],
    scratch_types=[
        pltpu.VMEM((_N,), jnp.float32),
        pltpu.VMEM((_N,), jnp.float32),
        pltpu.VMEM((_B,), jnp.int32),
        pltpu.VMEM((_BPW * 16,), jnp.float32),
        pltpu.VMEM((16,), jnp.float32),
        pltpu.SemaphoreType.DMA,
        pltpu.SemaphoreType.DMA,
    ],
    compiler_params=pltpu.CompilerParams(needs_layout_passes=False),
)
def _sc_stage(o1, o2, o3, o4, o5, o6, o7, o8, mim, tgt_hbm,
              marg_out, part_out,
              buf0, buf1, tgt_v, marg_v, pm_v, sem0, sem1):
    refs = [o1, o2, o3, o4, o5, o6, o7, o8, mim]
    wid = lax.axis_index("c") * _NS + lax.axis_index("s")
    b0 = wid * _BPW

    pltpu.sync_copy(tgt_hbm, tgt_v)

    rows = [(a, bi) for bi in range(_BPW) for a in range(_NA)]
    bufs = [buf0, buf1]
    sems = [sem0, sem1]

    def start(k):
        a, bi = rows[k]
        return pltpu.async_copy(refs[a].at[b0 + bi], bufs[k % 2], sems[k % 2])

    pending = start(0)
    pm = _NEG
    neg_vec = jnp.full((16,), _NEG)
    lane = lax.iota(jnp.int32, 16)
    marg_vec = jnp.zeros((16,), jnp.float32)

    for k, (a, bi) in enumerate(rows):
        buf = bufs[k % 2]
        cur = pending
        if k + 1 < len(rows):
            pending = start(k + 1)

        t_all = plsc.load_gather(tgt_v, [jnp.full((16,), b0 + bi, jnp.int32)])
        if a == 0:
            marg_vec = jnp.zeros((16,), jnp.float32)

        cur.wait()

        def mbody(i, accs, buf=buf):
            a0, a1, a2, a3 = accs
            base = i * 256
            acc = [a0, a1, a2, a3]
            for u in range(16):
                v = buf[pl.ds(base + u * 16, 16)]
                acc[u % 4] = jnp.maximum(acc[u % 4], v)
            return tuple(acc)

        a0, a1, a2, a3 = lax.fori_loop(
            0, _VPR // 16, mbody, (neg_vec, neg_vec, neg_vec, neg_vec))
        macc = jnp.maximum(jnp.maximum(a0, a1), jnp.maximum(a2, a3))
        m = jnp.max(macc)

        l0_vec = t_all & 15
        col_base = l0_vec + lane * 16
        cacc = neg_vec
        for i in range(16):
            idx = col_base + i * 256
            g = plsc.load_gather(buf, [idx])
            cacc = jnp.maximum(cacc, jnp.where(idx == t_all, _NEG, g))
        macc_o = jnp.where(lane == l0_vec, _NEG, macc)
        me = jnp.maximum(jnp.max(macc_o), jnp.max(cacc))

        if a < 8:
            pm = jnp.maximum(pm, m)

        marg_vec = jnp.where(lane == a, m - me, marg_vec)
        if a == _NA - 1:
            marg_v[pl.ds(bi * 16, 16)] = marg_vec

    pm_v[...] = jnp.full((16,), pm)
    pltpu.sync_copy(marg_v, marg_out.at[pl.ds(wid * (_BPW * 16), _BPW * 16)])
    pltpu.sync_copy(pm_v, part_out.at[pl.ds(wid * 16, 16)])


def _tc_body(marg_ref, part_ref, thr_ref, mx_ref):
    x = marg_ref[...]
    lanes = lax.broadcasted_iota(jnp.int32, (_B, 16), 1)
    valid = lanes < _NA
    logits = x * jnp.float32(1.0 / _T)
    mrow = jnp.max(jnp.where(valid, logits, jnp.float32(-1e30)),
                   axis=1, keepdims=True)
    e = jnp.where(valid, jnp.exp(logits - mrow), jnp.float32(0.0))
    s = jnp.sum(e, axis=1, keepdims=True)
    out = e / s
    thr_ref[...] = out[:, :_NA]
    mx_ref[...] = jnp.full((1, 1), jnp.max(part_ref[...]))


_tc_stage = pl.pallas_call(
    _tc_body,
    out_shape=(
        jax.ShapeDtypeStruct((_B, _NA), jnp.float32),
        jax.ShapeDtypeStruct((1, 1), jnp.float32),
    ),
)


def kernel(outputs1, outputs2, outputs3, outputs4, outputs5, outputs6,
           outputs7, outputs8, mimic, targets, n_test):
    marg_flat, parts = _sc_stage(
        outputs1, outputs2, outputs3, outputs4, outputs5, outputs6,
        outputs7, outputs8, mimic, targets.astype(jnp.int32))
    thr, mx = _tc_stage(marg_flat.reshape(_B, 16), parts.reshape(_NW, 16))
    return mx.reshape(()), thr


# speedup vs baseline: 1.1850x; 1.1850x over previous
import functools

import jax
import jax.numpy as jnp
from jax import lax
from jax.experimental import pallas as pl
from jax.experimental.pallas import tpu as pltpu
from jax.experimental.pallas import tpu_sc as plsc

_B = 128
_N = 4096
_T = 2.0
_NC = 2
_NS = 16
_NW = _NC * _NS
_BPW = _B // _NW
_NA = 9
_VPR = _N // 16
_NEG = float("-inf")


@functools.partial(
    pl.kernel,
    mesh=plsc.VectorSubcoreMesh(core_axis_name="c", subcore_axis_name="s"),
    out_type=[
        jax.ShapeDtypeStruct((_B * 16,), jnp.float32),
        jax.ShapeDtypeStruct((_NW * 16,), jnp.float32),
    ],
    scratch_types=[
        pltpu.VMEM((_BPW, _N), jnp.float32),
        pltpu.VMEM((_BPW, _N), jnp.float32),
        pltpu.VMEM((_B,), jnp.int32),
        pltpu.VMEM((_BPW * 16,), jnp.float32),
        pltpu.VMEM((16,), jnp.float32),
        pltpu.SemaphoreType.DMA,
        pltpu.SemaphoreType.DMA,
    ],
    compiler_params=pltpu.CompilerParams(needs_layout_passes=False),
)
def _sc_stage(o1, o2, o3, o4, o5, o6, o7, o8, mim, tgt_hbm,
              marg_out, part_out,
              buf0, buf1, tgt_v, marg_v, pm_v, sem0, sem1):
    refs = [o1, o2, o3, o4, o5, o6, o7, o8, mim]
    wid = lax.axis_index("c") * _NS + lax.axis_index("s")
    b0 = wid * _BPW

    pltpu.sync_copy(tgt_hbm, tgt_v)

    bufs = [buf0, buf1]
    sems = [sem0, sem1]

    def start(a):
        return pltpu.async_copy(refs[a].at[pl.ds(b0, _BPW)], bufs[a % 2],
                                sems[a % 2])

    pending = start(0)
    pm = _NEG
    neg_vec = jnp.full((16,), _NEG)
    lane = lax.iota(jnp.int32, 16)
    t_alls = [
        plsc.load_gather(tgt_v, [jnp.full((16,), b0 + bi, jnp.int32)])
        for bi in range(_BPW)
    ]
    marg_vecs = [jnp.zeros((16,), jnp.float32) for _ in range(_BPW)]

    for a in range(_NA):
        buf = bufs[a % 2]
        cur = pending
        if a + 1 < _NA:
            pending = start(a + 1)
        cur.wait()

        for bi in range(_BPW):
            t_all = t_alls[bi]
            bi_vec = jnp.full((16,), bi, jnp.int32)

            def mbody(i, accs, buf=buf, bi=bi):
                a0, a1, a2, a3 = accs
                base = i * 256
                acc = [a0, a1, a2, a3]
                for u in range(16):
                    v = buf[bi, pl.ds(base + u * 16, 16)]
                    acc[u % 4] = jnp.maximum(acc[u % 4], v)
                return tuple(acc)

            a0, a1, a2, a3 = lax.fori_loop(
                0, _VPR // 16, mbody, (neg_vec, neg_vec, neg_vec, neg_vec))
            macc = jnp.maximum(jnp.maximum(a0, a1), jnp.maximum(a2, a3))
            m = jnp.max(macc)
            v_t = jnp.max(plsc.load_gather(buf, [bi_vec, t_all]))

            def fix(buf=buf, bi_vec=bi_vec, t_all=t_all, macc=macc, m=m):
                l0_vec = t_all & 15
                col_base = l0_vec + lane * 16
                cacc = [neg_vec] * 4
                for i in range(16):
                    idx = col_base + i * 256
                    g = plsc.load_gather(buf, [bi_vec, idx])
                    cacc[i % 4] = jnp.maximum(
                        cacc[i % 4], jnp.where(idx == t_all, _NEG, g))
                call = jnp.maximum(jnp.maximum(cacc[0], cacc[1]),
                                   jnp.maximum(cacc[2], cacc[3]))
                macc_o = jnp.where(lane == l0_vec, _NEG, macc)
                me = jnp.maximum(jnp.max(macc_o), jnp.max(call))
                return m - me

            margin = lax.cond(v_t >= m, fix, lambda: jnp.float32(0.0))

            if a < 8:
                pm = jnp.maximum(pm, m)
            marg_vecs[bi] = jnp.where(lane == a, margin, marg_vecs[bi])

    for bi in range(_BPW):
        marg_v[pl.ds(bi * 16, 16)] = marg_vecs[bi]

    pm_v[...] = jnp.full((16,), pm)
    pltpu.sync_copy(marg_v, marg_out.at[pl.ds(wid * (_BPW * 16), _BPW * 16)])
    pltpu.sync_copy(pm_v, part_out.at[pl.ds(wid * 16, 16)])


def _tc_body(marg_ref, part_ref, thr_ref, mx_ref):
    x = marg_ref[...]
    lanes = lax.broadcasted_iota(jnp.int32, (_B, 16), 1)
    valid = lanes < _NA
    logits = x * jnp.float32(1.0 / _T)
    mrow = jnp.max(jnp.where(valid, logits, jnp.float32(-1e30)),
                   axis=1, keepdims=True)
    e = jnp.where(valid, jnp.exp(logits - mrow), jnp.float32(0.0))
    s = jnp.sum(e, axis=1, keepdims=True)
    out = e / s
    thr_ref[...] = out[:, :_NA]
    mx_ref[...] = jnp.full((1, 1), jnp.max(part_ref[...]))


_tc_stage = pl.pallas_call(
    _tc_body,
    out_shape=(
        jax.ShapeDtypeStruct((_B, _NA), jnp.float32),
        jax.ShapeDtypeStruct((1, 1), jnp.float32),
    ),
)


def kernel(outputs1, outputs2, outputs3, outputs4, outputs5, outputs6,
           outputs7, outputs8, mimic, targets, n_test):
    marg_flat, parts = _sc_stage(
        outputs1, outputs2, outputs3, outputs4, outputs5, outputs6,
        outputs7, outputs8, mimic, targets.astype(jnp.int32))
    thr, mx = _tc_stage(marg_flat.reshape(_B, 16), parts.reshape(_NW, 16))
    return mx.reshape(()), thr


# speedup vs baseline: 1.2172x; 1.0272x over previous
import functools

import jax
import jax.numpy as jnp
from jax import lax
from jax.experimental import pallas as pl
from jax.experimental.pallas import tpu as pltpu
from jax.experimental.pallas import tpu_sc as plsc

_B = 128
_N = 4096
_T = 2.0
_NC = 2
_NS = 16
_NW = _NC * _NS
_BPW = _B // _NW
_NA = 9
_VPR = _N // 16
_NEG = float("-inf")


@functools.partial(
    pl.kernel,
    mesh=plsc.VectorSubcoreMesh(core_axis_name="c", subcore_axis_name="s"),
    out_type=[
        jax.ShapeDtypeStruct((_B, 16), jnp.float32),
        jax.ShapeDtypeStruct((_NW, 16), jnp.float32),
    ],
    scratch_types=[
        pltpu.VMEM((_BPW, _N), jnp.float32),
        pltpu.VMEM((_BPW, _N), jnp.float32),
        pltpu.VMEM((_B,), jnp.int32),
        pltpu.VMEM((_BPW, 16), jnp.float32),
        pltpu.VMEM((1, 16), jnp.float32),
        pltpu.SemaphoreType.DMA,
        pltpu.SemaphoreType.DMA,
    ],
    compiler_params=pltpu.CompilerParams(needs_layout_passes=False),
)
def _sc_stage(o1, o2, o3, o4, o5, o6, o7, o8, mim, tgt_hbm,
              marg_out, part_out,
              buf0, buf1, tgt_v, marg_v, pm_v, sem0, sem1):
    refs = [o1, o2, o3, o4, o5, o6, o7, o8, mim]
    wid = lax.axis_index("c") * _NS + lax.axis_index("s")
    b0 = wid * _BPW

    pltpu.sync_copy(tgt_hbm, tgt_v)

    bufs = [buf0, buf1]
    sems = [sem0, sem1]

    def start(a):
        return pltpu.async_copy(refs[a].at[pl.ds(b0, _BPW)], bufs[a % 2],
                                sems[a % 2])

    pending = start(0)
    pm = _NEG
    neg_vec = jnp.full((16,), _NEG)
    lane = lax.iota(jnp.int32, 16)
    t_alls = [
        plsc.load_gather(tgt_v, [jnp.full((16,), b0 + bi, jnp.int32)])
        for bi in range(_BPW)
    ]
    marg_vecs = [jnp.zeros((16,), jnp.float32) for _ in range(_BPW)]

    for a in range(_NA):
        buf = bufs[a % 2]
        cur = pending
        if a + 1 < _NA:
            pending = start(a + 1)
        cur.wait()

        for bi in range(_BPW):
            t_all = t_alls[bi]
            bi_vec = jnp.full((16,), bi, jnp.int32)

            def mbody(i, accs, buf=buf, bi=bi):
                a0, a1, a2, a3 = accs
                base = i * 16
                acc = [a0, a1, a2, a3]
                for u in range(16):
                    v = buf[bi, pl.ds(base + u * 16, 16)]
                    acc[u % 4] = jnp.maximum(acc[u % 4], v)
                return tuple(acc)

            a0, a1, a2, a3 = plsc.parallel_loop(
                0, _VPR, step=16, unroll=2,
                carry=(neg_vec, neg_vec, neg_vec, neg_vec))(mbody)
            macc = jnp.maximum(jnp.maximum(a0, a1), jnp.maximum(a2, a3))
            m = jnp.max(macc)
            v_t = jnp.max(plsc.load_gather(buf, [bi_vec, t_all]))

            def fix(buf=buf, bi_vec=bi_vec, t_all=t_all, macc=macc, m=m):
                l0_vec = t_all & 15
                col_base = l0_vec + lane * 16
                cacc = [neg_vec] * 4
                for i in range(16):
                    idx = col_base + i * 256
                    g = plsc.load_gather(buf, [bi_vec, idx])
                    cacc[i % 4] = jnp.maximum(
                        cacc[i % 4], jnp.where(idx == t_all, _NEG, g))
                call = jnp.maximum(jnp.maximum(cacc[0], cacc[1]),
                                   jnp.maximum(cacc[2], cacc[3]))
                macc_o = jnp.where(lane == l0_vec, _NEG, macc)
                me = jnp.maximum(jnp.max(macc_o), jnp.max(call))
                return m - me

            margin = lax.cond(v_t >= m, fix, lambda: jnp.float32(0.0))

            if a < 8:
                pm = jnp.maximum(pm, m)
            marg_vecs[bi] = jnp.where(lane == a, margin, marg_vecs[bi])

    for bi in range(_BPW):
        marg_v[bi, :] = marg_vecs[bi]

    pm_v[0, :] = jnp.full((16,), pm)
    pltpu.sync_copy(marg_v, marg_out.at[pl.ds(b0, _BPW)])
    pltpu.sync_copy(pm_v, part_out.at[pl.ds(wid, 1)])


def _tc_body(marg_ref, part_ref, thr_ref, mx_ref):
    x = marg_ref[...]
    lanes = lax.broadcasted_iota(jnp.int32, (_B, 16), 1)
    valid = lanes < _NA
    logits = x * jnp.float32(1.0 / _T)
    mrow = jnp.max(jnp.where(valid, logits, jnp.float32(-1e30)),
                   axis=1, keepdims=True)
    e = jnp.where(valid, jnp.exp(logits - mrow), jnp.float32(0.0))
    s = jnp.sum(e, axis=1, keepdims=True)
    out = e / s
    thr_ref[...] = out[:, :_NA]
    mx_ref[...] = jnp.full((1, 1), jnp.max(part_ref[...]))


_tc_stage = pl.pallas_call(
    _tc_body,
    out_shape=(
        jax.ShapeDtypeStruct((_B, _NA), jnp.float32),
        jax.ShapeDtypeStruct((1, 1), jnp.float32),
    ),
)


def kernel(outputs1, outputs2, outputs3, outputs4, outputs5, outputs6,
           outputs7, outputs8, mimic, targets, n_test):
    marg, parts = _sc_stage(
        outputs1, outputs2, outputs3, outputs4, outputs5, outputs6,
        outputs7, outputs8, mimic, targets.astype(jnp.int32))
    thr, mx = _tc_stage(marg, parts)
    return mx.reshape(()), thr


# speedup vs baseline: 1.2233x; 1.0050x over previous
import functools

import jax
import jax.numpy as jnp
from jax import lax
from jax.experimental import pallas as pl
from jax.experimental.pallas import tpu as pltpu
from jax.experimental.pallas import tpu_sc as plsc

_B = 128
_N = 4096
_T = 2.0
_NC = 2
_NS = 16
_NW = _NC * _NS
_BPW = _B // _NW
_NA = 9
_VPR = _N // 16
_NEG = float("-inf")


@functools.partial(
    pl.kernel,
    mesh=plsc.VectorSubcoreMesh(core_axis_name="c", subcore_axis_name="s"),
    out_type=[
        jax.ShapeDtypeStruct((_B, 16), jnp.float32),
        jax.ShapeDtypeStruct((_NW, 16), jnp.float32),
    ],
    scratch_types=[
        pltpu.VMEM((_BPW, _N), jnp.float32),
        pltpu.VMEM((_BPW, _N), jnp.float32),
        pltpu.VMEM((_B,), jnp.int32),
        pltpu.VMEM((_BPW, 16), jnp.float32),
        pltpu.VMEM((1, 16), jnp.float32),
        pltpu.SemaphoreType.DMA,
        pltpu.SemaphoreType.DMA,
    ],
    compiler_params=pltpu.CompilerParams(needs_layout_passes=False),
)
def _sc_stage(o1, o2, o3, o4, o5, o6, o7, o8, mim, tgt_hbm,
              marg_out, part_out,
              buf0, buf1, tgt_v, marg_v, pm_v, sem0, sem1):
    refs = [o1, o2, o3, o4, o5, o6, o7, o8, mim]
    wid = lax.axis_index("c") * _NS + lax.axis_index("s")
    b0 = wid * _BPW

    pltpu.sync_copy(tgt_hbm, tgt_v)

    bufs = [buf0, buf1]
    sems = [sem0, sem1]

    def start(a):
        return pltpu.async_copy(refs[a].at[pl.ds(b0, _BPW)], bufs[a % 2],
                                sems[a % 2])

    pending = start(0)
    pm = _NEG
    neg_vec = jnp.full((16,), _NEG)
    lane = lax.iota(jnp.int32, 16)
    t_alls = [
        plsc.load_gather(tgt_v, [jnp.full((16,), b0 + bi, jnp.int32)])
        for bi in range(_BPW)
    ]
    marg_vecs = [jnp.zeros((16,), jnp.float32) for _ in range(_BPW)]

    for a in range(_NA):
        buf = bufs[a % 2]
        cur = pending
        if a + 1 < _NA:
            pending = start(a + 1)
        cur.wait()

        for bi in range(_BPW):
            t_all = t_alls[bi]
            bi_vec = jnp.full((16,), bi, jnp.int32)

            def mbody(i, accs, buf=buf, bi=bi):
                a0, a1, a2, a3 = accs
                base = i * 16
                acc = [a0, a1, a2, a3]
                for u in range(16):
                    v = buf[bi, pl.ds(base + u * 16, 16)]
                    acc[u % 4] = jnp.maximum(acc[u % 4], v)
                return tuple(acc)

            a0, a1, a2, a3 = plsc.parallel_loop(
                0, _VPR, step=16, unroll=2,
                carry=(neg_vec, neg_vec, neg_vec, neg_vec))(mbody)
            macc = jnp.maximum(jnp.maximum(a0, a1), jnp.maximum(a2, a3))
            m = jnp.max(macc)
            v_t = jnp.max(plsc.load_gather(buf, [bi_vec, t_all]))

            def fix(buf=buf, bi_vec=bi_vec, t_all=t_all, macc=macc, m=m):
                l0_vec = t_all & 15
                col_base = l0_vec + lane * 16
                cacc = [neg_vec] * 4
                for i in range(16):
                    idx = col_base + i * 256
                    g = plsc.load_gather(buf, [bi_vec, idx])
                    cacc[i % 4] = jnp.maximum(
                        cacc[i % 4], jnp.where(idx == t_all, _NEG, g))
                call = jnp.maximum(jnp.maximum(cacc[0], cacc[1]),
                                   jnp.maximum(cacc[2], cacc[3]))
                macc_o = jnp.where(lane == l0_vec, _NEG, macc)
                me = jnp.maximum(jnp.max(macc_o), jnp.max(call))
                return m - me

            margin = lax.cond(v_t >= m, fix, lambda: jnp.float32(0.0))

            if a < 8:
                pm = jnp.maximum(pm, m)
            marg_vecs[bi] = jnp.where(lane == a, margin, marg_vecs[bi])

    lmask = lane < _NA
    for bi in range(_BPW):
        logits = marg_vecs[bi] * (1.0 / _T)
        mx = jnp.max(jnp.where(lmask, logits, _NEG))
        e = jnp.where(lmask, jnp.exp(logits - mx), 0.0)
        marg_v[bi, :] = e / jnp.sum(e)

    pm_v[0, :] = jnp.full((16,), pm)
    pltpu.sync_copy(marg_v, marg_out.at[pl.ds(b0, _BPW)])
    pltpu.sync_copy(pm_v, part_out.at[pl.ds(wid, 1)])


def kernel(outputs1, outputs2, outputs3, outputs4, outputs5, outputs6,
           outputs7, outputs8, mimic, targets, n_test):
    thr16, parts = _sc_stage(
        outputs1, outputs2, outputs3, outputs4, outputs5, outputs6,
        outputs7, outputs8, mimic, targets.astype(jnp.int32))
    return jnp.max(parts), thr16[:, :_NA]


# speedup vs baseline: 1.4709x; 1.2024x over previous
import functools

import jax
import jax.numpy as jnp
from jax import lax
from jax.experimental import pallas as pl
from jax.experimental.pallas import tpu as pltpu
from jax.experimental.pallas import tpu_sc as plsc

_B = 128
_N = 4096
_T = 2.0
_NC = 2
_NS = 16
_NW = _NC * _NS
_BPW = _B // _NW
_NA = 9
_VPR = _N // 16
_NEG = float("-inf")


@functools.partial(
    pl.kernel,
    mesh=plsc.VectorSubcoreMesh(core_axis_name="c", subcore_axis_name="s"),
    out_type=[
        jax.ShapeDtypeStruct((_B, 16), jnp.float32),
        jax.ShapeDtypeStruct((_NW, 16), jnp.float32),
    ],
    scratch_types=[
        pltpu.VMEM((_BPW, _N), jnp.float32),
        pltpu.VMEM((_BPW, _N), jnp.float32),
        pltpu.VMEM((_B,), jnp.int32),
        pltpu.VMEM((_BPW, 16), jnp.float32),
        pltpu.VMEM((1, 16), jnp.float32),
        pltpu.SemaphoreType.DMA,
        pltpu.SemaphoreType.DMA,
    ],
    compiler_params=pltpu.CompilerParams(needs_layout_passes=False),
)
def _sc_stage(o1, o2, o3, o4, o5, o6, o7, o8, mim, tgt_hbm,
              marg_out, part_out,
              buf0, buf1, tgt_v, marg_v, pm_v, sem0, sem1):
    refs = [o1, o2, o3, o4, o5, o6, o7, o8, mim]
    wid = lax.axis_index("c") * _NS + lax.axis_index("s")
    b0 = wid * _BPW

    pltpu.sync_copy(tgt_hbm, tgt_v)

    bufs = [buf0, buf1]
    sems = [sem0, sem1]

    def start(a):
        return pltpu.async_copy(refs[a].at[pl.ds(b0, _BPW)], bufs[a % 2],
                                sems[a % 2])

    pending = start(0)
    pm = jnp.float32(_NEG)
    neg_vec = jnp.full((16,), _NEG)
    lane = lax.iota(jnp.int32, 16)
    zero_vec = jnp.zeros((16,), jnp.float32)
    lmask = lane < _NA

    for bi in range(_BPW):
        marg_v[bi, :] = zero_vec

    for a in range(_NA):
        buf = bufs[a % 2]
        cur = pending
        if a + 1 < _NA:
            pending = start(a + 1)
        cur.wait()

        def row_body(bi, pm, buf=buf, a=a):
            bi_vec = jnp.full((16,), bi, jnp.int32)
            t_all = plsc.load_gather(tgt_v, [jnp.full((16,), b0 + bi, jnp.int32)])

            def mbody(i, accs):
                a0, a1, a2, a3 = accs
                base = i * 16
                acc = [a0, a1, a2, a3]
                for u in range(16):
                    v = buf[bi, pl.ds(base + u * 16, 16)]
                    acc[u % 4] = jnp.maximum(acc[u % 4], v)
                return tuple(acc)

            a0, a1, a2, a3 = plsc.parallel_loop(
                0, _VPR, step=16, unroll=2,
                carry=(neg_vec, neg_vec, neg_vec, neg_vec))(mbody)
            macc = jnp.maximum(jnp.maximum(a0, a1), jnp.maximum(a2, a3))
            m = jnp.max(macc)
            v_t = jnp.max(plsc.load_gather(buf, [bi_vec, t_all]))

            def fix():
                l0_vec = t_all & 15
                col_base = l0_vec + lane * 16

                def fbody(i, caccs):
                    out = list(caccs)
                    for u in range(4):
                        idx = col_base + (i * 4 + u) * 256
                        g = plsc.load_gather(buf, [bi_vec, idx])
                        out[u] = jnp.maximum(
                            out[u], jnp.where(idx == t_all, _NEG, g))
                    return tuple(out)

                c0, c1, c2, c3 = lax.fori_loop(
                    0, 4, fbody, (neg_vec, neg_vec, neg_vec, neg_vec))
                call = jnp.maximum(jnp.maximum(c0, c1), jnp.maximum(c2, c3))
                macc_o = jnp.where(lane == l0_vec, _NEG, macc)
                me = jnp.maximum(jnp.max(macc_o), jnp.max(call))
                return m - me

            margin = lax.cond(v_t >= m, fix, lambda: jnp.float32(0.0))

            mv = marg_v[bi, :]
            marg_v[bi, :] = jnp.where(lane == a, margin, mv)
            if a < 8:
                pm = jnp.maximum(pm, m)
            return pm

        pm = lax.fori_loop(0, _BPW, row_body, pm)

    def sm_body(bi, c):
        logits = marg_v[bi, :] * (1.0 / _T)
        mx = jnp.max(jnp.where(lmask, logits, _NEG))
        e = jnp.where(lmask, jnp.exp(logits - mx), 0.0)
        marg_v[bi, :] = e / jnp.sum(e)
        return c

    lax.fori_loop(0, _BPW, sm_body, jnp.int32(0))

    pm_v[0, :] = jnp.full((16,), pm)
    pltpu.sync_copy(marg_v, marg_out.at[pl.ds(b0, _BPW)])
    pltpu.sync_copy(pm_v, part_out.at[pl.ds(wid, 1)])


def kernel(outputs1, outputs2, outputs3, outputs4, outputs5, outputs6,
           outputs7, outputs8, mimic, targets, n_test):
    thr16, parts = _sc_stage(
        outputs1, outputs2, outputs3, outputs4, outputs5, outputs6,
        outputs7, outputs8, mimic, targets.astype(jnp.int32))
    return jnp.max(parts), thr16[:, :_NA]


# speedup vs baseline: 1.7019x; 1.1571x over previous
import functools

import jax
import jax.numpy as jnp
from jax import lax
from jax.experimental import pallas as pl
from jax.experimental.pallas import tpu as pltpu
from jax.experimental.pallas import tpu_sc as plsc

_B = 128
_N = 4096
_T = 2.0
_NC = 2
_NS = 16
_NW = _NC * _NS
_BPW = _B // _NW
_NA = 9
_VPR = _N // 16
_NSC = 4
_NBLK = 8
_BLK = _N // _NBLK
_NEG = float("-inf")


@functools.partial(
    pl.kernel,
    mesh=plsc.VectorSubcoreMesh(core_axis_name="c", subcore_axis_name="s"),
    out_type=[
        jax.ShapeDtypeStruct((_B, 16), jnp.float32),
        jax.ShapeDtypeStruct((_NW, 16), jnp.float32),
    ],
    scratch_types=[
        pltpu.VMEM((_BPW, _N), jnp.float32),
        pltpu.VMEM((_BPW, _N), jnp.float32),
        pltpu.VMEM((_B,), jnp.int32),
        pltpu.VMEM((_BPW, 16), jnp.float32),
        pltpu.VMEM((1, 16), jnp.float32),
        pltpu.SemaphoreType.DMA,
        pltpu.SemaphoreType.DMA,
    ],
    compiler_params=pltpu.CompilerParams(needs_layout_passes=False),
)
def _sc_stage(o1, o2, o3, o4, tgt_hbm,
              marg_out, part_out,
              buf0, buf1, tgt_v, marg_v, pm_v, sem0, sem1):
    refs = [o1, o2, o3, o4]
    wid = lax.axis_index("c") * _NS + lax.axis_index("s")
    b0 = wid * _BPW

    pltpu.sync_copy(tgt_hbm, tgt_v)

    bufs = [buf0, buf1]
    sems = [sem0, sem1]

    def start(a):
        return pltpu.async_copy(refs[a].at[pl.ds(b0, _BPW)], bufs[a % 2],
                                sems[a % 2])

    pending = start(0)
    pm = jnp.float32(_NEG)
    neg_vec = jnp.full((16,), _NEG)
    lane = lax.iota(jnp.int32, 16)
    zero_vec = jnp.zeros((16,), jnp.float32)

    for bi in range(_BPW):
        marg_v[bi, :] = zero_vec

    for a in range(_NSC):
        buf = bufs[a % 2]
        cur = pending
        if a + 1 < _NSC:
            pending = start(a + 1)
        cur.wait()

        def row_body(bi, pm, buf=buf, a=a):
            bi_vec = jnp.full((16,), bi, jnp.int32)
            t_all = plsc.load_gather(tgt_v, [jnp.full((16,), b0 + bi, jnp.int32)])

            def mbody(i, accs):
                a0, a1, a2, a3 = accs
                base = i * 16
                acc = [a0, a1, a2, a3]
                for u in range(16):
                    v = buf[bi, pl.ds(base + u * 16, 16)]
                    acc[u % 4] = jnp.maximum(acc[u % 4], v)
                return tuple(acc)

            a0, a1, a2, a3 = plsc.parallel_loop(
                0, _VPR, step=16, unroll=2,
                carry=(neg_vec, neg_vec, neg_vec, neg_vec))(mbody)
            macc = jnp.maximum(jnp.maximum(a0, a1), jnp.maximum(a2, a3))
            m = jnp.max(macc)
            v_t = jnp.max(plsc.load_gather(buf, [bi_vec, t_all]))

            def fix():
                l0_vec = t_all & 15
                col_base = l0_vec + lane * 16

                def fbody(i, caccs):
                    out = list(caccs)
                    for u in range(4):
                        idx = col_base + (i * 4 + u) * 256
                        g = plsc.load_gather(buf, [bi_vec, idx])
                        out[u] = jnp.maximum(
                            out[u], jnp.where(idx == t_all, _NEG, g))
                    return tuple(out)

                c0, c1, c2, c3 = lax.fori_loop(
                    0, 4, fbody, (neg_vec, neg_vec, neg_vec, neg_vec))
                call = jnp.maximum(jnp.maximum(c0, c1), jnp.maximum(c2, c3))
                macc_o = jnp.where(lane == l0_vec, _NEG, macc)
                me = jnp.maximum(jnp.max(macc_o), jnp.max(call))
                return m - me

            margin = lax.cond(v_t >= m, fix, lambda: jnp.float32(0.0))

            mv = marg_v[bi, :]
            marg_v[bi, :] = jnp.where(lane == a, margin, mv)
            return jnp.maximum(pm, m)

        pm = lax.fori_loop(0, _BPW, row_body, pm)

    pm_v[0, :] = jnp.full((16,), pm)
    pltpu.sync_copy(marg_v, marg_out.at[pl.ds(b0, _BPW)])
    pltpu.sync_copy(pm_v, part_out.at[pl.ds(wid, 1)])


def _tc_marg_body(o5r, o6r, o7r, o8r, mimr, tgtr, marg_ref, pm_ref, acc_ref):
    i = pl.program_id(0)
    tcol = tgtr[...]
    col = lax.broadcasted_iota(jnp.int32, (_B, _BLK), 1) + i * _BLK
    hit = col == tcol
    neg = jnp.float32(_NEG)
    ms, mes = [], []
    for r in (o5r, o6r, o7r, o8r, mimr):
        x = r[...]
        ms.append(jnp.max(x, axis=1, keepdims=True))
        mes.append(jnp.max(jnp.where(hit, neg, x), axis=1, keepdims=True))
    new = jnp.concatenate(
        ms + mes + [jnp.full((_B, 16 - 2 * 5), _NEG)], axis=1)
    prev = jnp.where(i == 0, jnp.full((_B, 16), _NEG), acc_ref[...])
    upd = jnp.maximum(prev, new)
    acc_ref[...] = upd

    @pl.when(i == _NBLK - 1)
    def _():
        m5 = upd[:, 0:5]
        me5 = upd[:, 5:10]
        marg_ref[...] = jnp.concatenate(
            [jnp.zeros((_B, _NSC)), m5 - me5, jnp.zeros((_B, 16 - _NSC - 5))],
            axis=1)
        pm_ref[...] = jnp.full((1, 1), jnp.max(upd[:, 0:4]))


_tc_marg = pl.pallas_call(
    _tc_marg_body,
    grid=(_NBLK,),
    in_specs=[pl.BlockSpec((_B, _BLK), lambda i: (0, i))] * 5
    + [pl.BlockSpec((_B, 1), lambda i: (0, 0))],
    out_specs=(pl.BlockSpec((_B, 16), lambda i: (0, 0)),
               pl.BlockSpec((1, 1), lambda i: (0, 0))),
    out_shape=(jax.ShapeDtypeStruct((_B, 16), jnp.float32),
               jax.ShapeDtypeStruct((1, 1), jnp.float32)),
    scratch_shapes=[pltpu.VMEM((_B, 16), jnp.float32)],
)


def _fin_body(msc, mtc, psc, ptc, thr_ref, mx_ref):
    margins = msc[...] + mtc[...]
    lanes = lax.broadcasted_iota(jnp.int32, (_B, 16), 1)
    valid = lanes < _NA
    logits = margins * jnp.float32(1.0 / _T)
    mrow = jnp.max(jnp.where(valid, logits, jnp.float32(-1e30)),
                   axis=1, keepdims=True)
    e = jnp.where(valid, jnp.exp(logits - mrow), jnp.float32(0.0))
    thr_ref[...] = (e / jnp.sum(e, axis=1, keepdims=True))[:, :_NA]
    mx_ref[...] = jnp.full((1, 1),
                           jnp.maximum(jnp.max(psc[...]), jnp.max(ptc[...])))


_fin = pl.pallas_call(
    _fin_body,
    out_shape=(jax.ShapeDtypeStruct((_B, _NA), jnp.float32),
               jax.ShapeDtypeStruct((1, 1), jnp.float32)),
)


def kernel(outputs1, outputs2, outputs3, outputs4, outputs5, outputs6,
           outputs7, outputs8, mimic, targets, n_test):
    tgt32 = targets.astype(jnp.int32)
    marg_sc, part_sc = _sc_stage(outputs1, outputs2, outputs3, outputs4,
                                 tgt32)
    marg_tc, pm_tc = _tc_marg(outputs5, outputs6, outputs7, outputs8, mimic,
                              tgt32.reshape(_B, 1))
    thr, mx = _fin(marg_sc, marg_tc, part_sc, pm_tc)
    return mx.reshape(()), thr


# speedup vs baseline: 1.7204x; 1.0109x over previous
import functools

import jax
import jax.numpy as jnp
from jax import lax
from jax.experimental import pallas as pl
from jax.experimental.pallas import tpu as pltpu
from jax.experimental.pallas import tpu_sc as plsc

_B = 128
_N = 4096
_T = 2.0
_NC = 2
_NS = 16
_NW = _NC * _NS
_BPW = _B // _NW
_NA = 9
_VPR = _N // 16
_NSC = 4
_NBLK = 8
_BLK = _N // _NBLK
_NEG = float("-inf")


@functools.partial(
    pl.kernel,
    mesh=plsc.VectorSubcoreMesh(core_axis_name="c", subcore_axis_name="s"),
    out_type=[
        jax.ShapeDtypeStruct((_B, 16), jnp.float32),
        jax.ShapeDtypeStruct((_NW, 16), jnp.float32),
    ],
    scratch_types=[
        pltpu.VMEM((_BPW, _N), jnp.float32),
        pltpu.VMEM((_BPW, _N), jnp.float32),
        pltpu.VMEM((_B,), jnp.int32),
        pltpu.VMEM((_BPW, 16), jnp.float32),
        pltpu.VMEM((1, 16), jnp.float32),
        pltpu.SemaphoreType.DMA,
        pltpu.SemaphoreType.DMA,
    ],
    compiler_params=pltpu.CompilerParams(needs_layout_passes=False,
                                         skip_device_barrier=True),
)
def _sc_stage(o1, o2, o3, o4, tgt_hbm,
              marg_out, part_out,
              buf0, buf1, tgt_v, marg_v, pm_v, sem0, sem1):
    refs = [o1, o2, o3, o4]
    wid = lax.axis_index("c") * _NS + lax.axis_index("s")
    b0 = wid * _BPW

    pltpu.sync_copy(tgt_hbm, tgt_v)

    bufs = [buf0, buf1]
    sems = [sem0, sem1]

    def start(a):
        return pltpu.async_copy(refs[a].at[pl.ds(b0, _BPW)], bufs[a % 2],
                                sems[a % 2])

    pending = start(0)
    pm = jnp.float32(_NEG)
    neg_vec = jnp.full((16,), _NEG)
    lane = lax.iota(jnp.int32, 16)
    zero_vec = jnp.zeros((16,), jnp.float32)

    for bi in range(_BPW):
        marg_v[bi, :] = zero_vec

    for a in range(_NSC):
        buf = bufs[a % 2]
        cur = pending
        if a + 1 < _NSC:
            pending = start(a + 1)
        cur.wait()

        def row_body(bi, pm, buf=buf, a=a):
            bi_vec = jnp.full((16,), bi, jnp.int32)
            t_all = plsc.load_gather(tgt_v, [jnp.full((16,), b0 + bi, jnp.int32)])

            def mbody(i, accs):
                a0, a1, a2, a3 = accs
                base = i * 16
                acc = [a0, a1, a2, a3]
                for u in range(16):
                    v = buf[bi, pl.ds(base + u * 16, 16)]
                    acc[u % 4] = jnp.maximum(acc[u % 4], v)
                return tuple(acc)

            a0, a1, a2, a3 = plsc.parallel_loop(
                0, _VPR, step=16, unroll=2,
                carry=(neg_vec, neg_vec, neg_vec, neg_vec))(mbody)
            macc = jnp.maximum(jnp.maximum(a0, a1), jnp.maximum(a2, a3))
            m = jnp.max(macc)
            v_t = plsc.load_gather(buf, [bi_vec, t_all])[0]

            def fix():
                l0_vec = t_all & 15
                col_base = l0_vec + lane * 16

                def fbody(i, caccs):
                    out = list(caccs)
                    for u in range(4):
                        idx = col_base + (i * 4 + u) * 256
                        g = plsc.load_gather(buf, [bi_vec, idx])
                        out[u] = jnp.maximum(
                            out[u], jnp.where(idx == t_all, _NEG, g))
                    return tuple(out)

                c0, c1, c2, c3 = lax.fori_loop(
                    0, 4, fbody, (neg_vec, neg_vec, neg_vec, neg_vec))
                call = jnp.maximum(jnp.maximum(c0, c1), jnp.maximum(c2, c3))
                macc_o = jnp.where(lane == l0_vec, _NEG, macc)
                me = jnp.maximum(jnp.max(macc_o), jnp.max(call))
                return m - me

            margin = lax.cond(v_t >= m, fix, lambda: jnp.float32(0.0))

            mv = marg_v[bi, :]
            marg_v[bi, :] = jnp.where(lane == a, margin, mv)
            return jnp.maximum(pm, m)

        pm = lax.fori_loop(0, _BPW, row_body, pm)

    pm_v[0, :] = jnp.full((16,), pm)
    pltpu.sync_copy(marg_v, marg_out.at[pl.ds(b0, _BPW)])
    pltpu.sync_copy(pm_v, part_out.at[pl.ds(wid, 1)])


def _tc_marg_body(o5r, o6r, o7r, o8r, mimr, tgtr, marg_ref, pm_ref, acc_ref):
    i = pl.program_id(0)
    tcol = tgtr[...]
    col = lax.broadcasted_iota(jnp.int32, (_B, _BLK), 1) + i * _BLK
    hit = col == tcol
    neg = jnp.float32(_NEG)
    ms, mes = [], []
    for r in (o5r, o6r, o7r, o8r, mimr):
        x = r[...]
        ms.append(jnp.max(x, axis=1, keepdims=True))
        mes.append(jnp.max(jnp.where(hit, neg, x), axis=1, keepdims=True))
    new = jnp.concatenate(
        ms + mes + [jnp.full((_B, 16 - 2 * 5), _NEG)], axis=1)
    prev = jnp.where(i == 0, jnp.full((_B, 16), _NEG), acc_ref[...])
    upd = jnp.maximum(prev, new)
    acc_ref[...] = upd

    @pl.when(i == _NBLK - 1)
    def _():
        m5 = upd[:, 0:5]
        me5 = upd[:, 5:10]
        marg_ref[...] = jnp.concatenate(
            [jnp.zeros((_B, _NSC)), m5 - me5, jnp.zeros((_B, 16 - _NSC - 5))],
            axis=1)
        pm_ref[...] = jnp.full((1, 1), jnp.max(upd[:, 0:4]))


_tc_marg = pl.pallas_call(
    _tc_marg_body,
    grid=(_NBLK,),
    in_specs=[pl.BlockSpec((_B, _BLK), lambda i: (0, i))] * 5
    + [pl.BlockSpec((_B, 1), lambda i: (0, 0))],
    out_specs=(pl.BlockSpec((_B, 16), lambda i: (0, 0)),
               pl.BlockSpec((1, 1), lambda i: (0, 0))),
    out_shape=(jax.ShapeDtypeStruct((_B, 16), jnp.float32),
               jax.ShapeDtypeStruct((1, 1), jnp.float32)),
    scratch_shapes=[pltpu.VMEM((_B, 16), jnp.float32)],
)


def _fin_body(msc, mtc, psc, ptc, thr_ref, mx_ref):
    margins = msc[...] + mtc[...]
    lanes = lax.broadcasted_iota(jnp.int32, (_B, 16), 1)
    valid = lanes < _NA
    logits = margins * jnp.float32(1.0 / _T)
    mrow = jnp.max(jnp.where(valid, logits, jnp.float32(-1e30)),
                   axis=1, keepdims=True)
    e = jnp.where(valid, jnp.exp(logits - mrow), jnp.float32(0.0))
    thr_ref[...] = (e / jnp.sum(e, axis=1, keepdims=True))[:, :_NA]
    mx_ref[...] = jnp.full((1, 1),
                           jnp.maximum(jnp.max(psc[...]), jnp.max(ptc[...])))


_fin = pl.pallas_call(
    _fin_body,
    out_shape=(jax.ShapeDtypeStruct((_B, _NA), jnp.float32),
               jax.ShapeDtypeStruct((1, 1), jnp.float32)),
)


def kernel(outputs1, outputs2, outputs3, outputs4, outputs5, outputs6,
           outputs7, outputs8, mimic, targets, n_test):
    tgt32 = targets.astype(jnp.int32)
    marg_sc, part_sc = _sc_stage(outputs1, outputs2, outputs3, outputs4,
                                 tgt32)
    marg_tc, pm_tc = _tc_marg(outputs5, outputs6, outputs7, outputs8, mimic,
                              tgt32.reshape(_B, 1))
    thr, mx = _fin(marg_sc, marg_tc, part_sc, pm_tc)
    return mx.reshape(()), thr
